# baseline (speedup 1.0000x reference)
import functools

import jax
import jax.numpy as jnp
from jax import lax
from jax.experimental import pallas as pl
from jax.experimental.pallas import tpu as pltpu

_BETA = 0.9
_FREQ_BETA = 0.95
_TARGET_FREQUENCY = 0.1
_REFRACTORY_INPUT_GAIN = -0.3

_N = 4096
_RB = 4
_NB = _N // _RB


def _lateral_body(order_ref, spv_ref, w_ref, out_ref):
    i = pl.program_id(0)

    @pl.when(i == 0)
    def _():
        out_ref[...] = jnp.zeros_like(out_ref)

    blk = order_ref[i]
    live = (blk == i).astype(jnp.float32)
    for j in range(_RB):
        v = spv_ref[blk * _RB + j].astype(jnp.float32) * live
        out_ref[j] += w_ref[0, j] * v


@jax.jit
def _lateral_partials(order, sp_i32, weights4d):
    grid_spec = pltpu.PrefetchScalarGridSpec(
        num_scalar_prefetch=2,
        grid=(_NB,),
        in_specs=[
            pl.BlockSpec(
                (1, _RB, 32, 128),
                lambda i, order_ref, spv_ref: (order_ref[i], 0, 0, 0),
            ),
        ],
        out_specs=pl.BlockSpec(
            (_RB, 32, 128),
            lambda i, order_ref, spv_ref: (0, 0, 0),
        ),
    )
    return pl.pallas_call(
        _lateral_body,
        grid_spec=grid_spec,
        out_shape=jax.ShapeDtypeStruct((_RB, 32, 128), jnp.float32),
    )(order, sp_i32, weights4d)


def _finish_body(part_ref, x_ref, act_ref, gain_ref, thr_ref, freq_ref,
                 ns_ref, act_o_ref, thr_o_ref, gain_o_ref, freq_o_ref):
    lat = jnp.sum(part_ref[...], axis=0)
    gain = gain_ref[...]
    gain = gain + (1.0 - gain) * 0.2
    xt = x_ref[...] + lat
    act = _BETA * act_ref[...] + xt * gain + 0.05
    thr = thr_ref[...]
    ns = act > thr
    nsf = ns.astype(jnp.float32)
    freq = _FREQ_BETA * freq_ref[...] + (1.0 - _FREQ_BETA) * nsf
    thr = jnp.where(freq > _TARGET_FREQUENCY, thr + 0.05, thr)
    thr = jnp.where(freq < _TARGET_FREQUENCY, thr / 1.05, thr)
    gain = jnp.where(ns, _REFRACTORY_INPUT_GAIN, gain)
    act = jnp.where(ns, 0.0, act)
    ns_ref[...] = nsf
    act_o_ref[...] = act
    thr_o_ref[...] = thr
    gain_o_ref[...] = gain
    freq_o_ref[...] = freq


@jax.jit
def _finish(partials, x, act, gain, thr, freq):
    out = jax.ShapeDtypeStruct((32, 128), jnp.float32)
    return pl.pallas_call(
        _finish_body,
        out_shape=(out, out, out, out, out),
    )(partials, x, act, gain, thr, freq)


def kernel(x, activation, input_gain, spikes, threshold, freq_act,
           lateral_weights):
    shape = x.shape

    sp_flags = spikes.reshape(_NB, _RB)
    nonempty = jnp.any(sp_flags, axis=1)
    iota = jnp.arange(_NB, dtype=jnp.int32)
    order = jnp.maximum(
        jax.lax.cummax(jnp.where(nonempty, iota, -1)), 0
    ).astype(jnp.int32)

    sp_i32 = spikes.reshape(-1).astype(jnp.int32)
    weights4d = lateral_weights.reshape(_NB, _RB, 32, 128)

    partials = _lateral_partials(order, sp_i32, weights4d)
    nsf, act, thr, gain, freq = _finish(
        partials,
        x.reshape(32, 128),
        activation.reshape(32, 128),
        input_gain.reshape(32, 128),
        threshold.reshape(32, 128),
        freq_act.reshape(32, 128),
    )
    return (
        nsf.reshape(shape).astype(bool),
        act.reshape(shape),
        thr.reshape(shape),
        gain.reshape(shape),
        freq.reshape(shape),
    )


# speedup vs baseline: 2.4561x; 2.4561x over previous
import functools

import jax
import jax.numpy as jnp
from jax import lax
from jax.experimental import pallas as pl
from jax.experimental.pallas import tpu as pltpu

_BETA = 0.9
_FREQ_BETA = 0.95
_TARGET_FREQUENCY = 0.1
_REFRACTORY_INPUT_GAIN = -0.3

_N = 4096
_RB = 4
_NB = _N // _RB


_K = 8
_STEPS = _NB // _K


def _lateral_body(order_ref, spv_ref, *refs):
    w_refs, out_ref = refs[:_K], refs[_K]
    i = pl.program_id(0)

    @pl.when(i == 0)
    def _():
        out_ref[...] = jnp.zeros_like(out_ref)

    for k in range(_K):
        blk = order_ref[i * _K + k]
        for j in range(_RB):
            v = spv_ref[blk * _RB + j].astype(jnp.float32)
            out_ref[j] += w_refs[k][0, j] * v


@jax.jit
def _lateral_partials(order, sp_i32, weights4d):
    def w_index_map(k):
        def index_map(i, order_ref, spv_ref):
            return (order_ref[i * _K + k], 0, 0, 0)
        return index_map

    grid_spec = pltpu.PrefetchScalarGridSpec(
        num_scalar_prefetch=2,
        grid=(_STEPS,),
        in_specs=[
            pl.BlockSpec((1, _RB, 32, 128), w_index_map(k))
            for k in range(_K)
        ],
        out_specs=pl.BlockSpec(
            (_RB, 32, 128),
            lambda i, order_ref, spv_ref: (0, 0, 0),
        ),
    )
    return pl.pallas_call(
        _lateral_body,
        grid_spec=grid_spec,
        out_shape=jax.ShapeDtypeStruct((_RB, 32, 128), jnp.float32),
    )(order, sp_i32, *([weights4d] * _K))


def _finish_body(part_ref, x_ref, act_ref, gain_ref, thr_ref, freq_ref,
                 ns_ref, act_o_ref, thr_o_ref, gain_o_ref, freq_o_ref):
    lat = jnp.sum(part_ref[...], axis=0)
    gain = gain_ref[...]
    gain = gain + (1.0 - gain) * 0.2
    xt = x_ref[...] + lat
    act = _BETA * act_ref[...] + xt * gain + 0.05
    thr = thr_ref[...]
    ns = act > thr
    nsf = ns.astype(jnp.float32)
    freq = _FREQ_BETA * freq_ref[...] + (1.0 - _FREQ_BETA) * nsf
    thr = jnp.where(freq > _TARGET_FREQUENCY, thr + 0.05, thr)
    thr = jnp.where(freq < _TARGET_FREQUENCY, thr / 1.05, thr)
    gain = jnp.where(ns, _REFRACTORY_INPUT_GAIN, gain)
    act = jnp.where(ns, 0.0, act)
    ns_ref[...] = nsf
    act_o_ref[...] = act
    thr_o_ref[...] = thr
    gain_o_ref[...] = gain
    freq_o_ref[...] = freq


@jax.jit
def _finish(partials, x, act, gain, thr, freq):
    out = jax.ShapeDtypeStruct((32, 128), jnp.float32)
    return pl.pallas_call(
        _finish_body,
        out_shape=(out, out, out, out, out),
    )(partials, x, act, gain, thr, freq)


def kernel(x, activation, input_gain, spikes, threshold, freq_act,
           lateral_weights):
    shape = x.shape

    sp_flags = spikes.reshape(_NB, _RB)
    nonempty = jnp.any(sp_flags, axis=1)
    order = jnp.argsort(~nonempty, stable=True).astype(jnp.int32)
    nn = jnp.sum(nonempty.astype(jnp.int32))
    tail_id = order[jnp.minimum(nn, _NB - 1)]
    steps = jnp.arange(_NB, dtype=jnp.int32)
    order = jnp.where(steps < nn, order, tail_id)

    sp_i32 = spikes.reshape(-1).astype(jnp.int32)
    weights4d = lateral_weights.reshape(_NB, _RB, 32, 128)

    partials = _lateral_partials(order, sp_i32, weights4d)
    nsf, act, thr, gain, freq = _finish(
        partials,
        x.reshape(32, 128),
        activation.reshape(32, 128),
        input_gain.reshape(32, 128),
        threshold.reshape(32, 128),
        freq_act.reshape(32, 128),
    )
    return (
        nsf.reshape(shape).astype(bool),
        act.reshape(shape),
        thr.reshape(shape),
        gain.reshape(shape),
        freq.reshape(shape),
    )


# speedup vs baseline: 6.8248x; 2.7787x over previous
import jax
import jax.numpy as jnp
from jax.experimental import pallas as pl
from jax.experimental.pallas import tpu as pltpu

_BETA = 0.9
_FREQ_BETA = 0.95
_TARGET_FREQUENCY = 0.1
_REFRACTORY_INPUT_GAIN = -0.3

_N = 4096
_PR = 128
_STEPS = _N // _PR


def _body(sp_ref, w_ref, x_ref, act_ref, gain_ref, thr_ref, freq_ref,
          ns_ref, act_o_ref, thr_o_ref, gain_o_ref, freq_o_ref, acc_ref):
    i = pl.program_id(0)

    @pl.when(i == 0)
    def _():
        acc_ref[...] = jnp.zeros_like(acc_ref)

    acc_ref[...] += jnp.dot(sp_ref[...], w_ref[...],
                            preferred_element_type=jnp.float32)

    @pl.when(i == _STEPS - 1)
    def _():
        lat = acc_ref[...]
        gain = gain_ref[...]
        gain = gain + (1.0 - gain) * 0.2
        xt = x_ref[...] + lat
        act = _BETA * act_ref[...] + xt * gain + 0.05
        thr = thr_ref[...]
        ns = act > thr
        nsf = ns.astype(jnp.float32)
        freq = _FREQ_BETA * freq_ref[...] + (1.0 - _FREQ_BETA) * nsf
        thr = jnp.where(freq > _TARGET_FREQUENCY, thr + 0.05, thr)
        thr = jnp.where(freq < _TARGET_FREQUENCY, thr / 1.05, thr)
        gain = jnp.where(ns, _REFRACTORY_INPUT_GAIN, gain)
        act = jnp.where(ns, 0.0, act)
        ns_ref[...] = nsf
        act_o_ref[...] = act
        thr_o_ref[...] = thr
        gain_o_ref[...] = gain
        freq_o_ref[...] = freq


@jax.jit
def _step(sp, weights, x, act, gain, thr, freq):
    flat = jax.ShapeDtypeStruct((1, _N), jnp.float32)
    state_spec = pl.BlockSpec((1, _N), lambda i: (0, 0))
    return pl.pallas_call(
        _body,
        grid=(_STEPS,),
        in_specs=[
            pl.BlockSpec((1, _PR), lambda i: (0, i)),
            pl.BlockSpec((_PR, _N), lambda i: (i, 0)),
            state_spec, state_spec, state_spec, state_spec, state_spec,
        ],
        out_specs=(state_spec,) * 5,
        out_shape=(flat,) * 5,
        scratch_shapes=[pltpu.VMEM((1, _N), jnp.float32)],
    )(sp, weights, x, act, gain, thr, freq)


def kernel(x, activation, input_gain, spikes, threshold, freq_act,
           lateral_weights):
    shape = x.shape
    sp = spikes.reshape(1, _N).astype(jnp.float32)
    nsf, act, thr, gain, freq = _step(
        sp,
        lateral_weights,
        x.reshape(1, _N),
        activation.reshape(1, _N),
        input_gain.reshape(1, _N),
        threshold.reshape(1, _N),
        freq_act.reshape(1, _N),
    )
    return (
        nsf.reshape(shape).astype(bool),
        act.reshape(shape),
        thr.reshape(shape),
        gain.reshape(shape),
        freq.reshape(shape),
    )


# speedup vs baseline: 8.1550x; 1.1949x over previous
import jax
import jax.numpy as jnp
from jax.experimental import pallas as pl
from jax.experimental.pallas import tpu as pltpu

_BETA = 0.9
_FREQ_BETA = 0.95
_TARGET_FREQUENCY = 0.1
_REFRACTORY_INPUT_GAIN = -0.3

_N = 4096
_PR = 256
_STEPS = _N // _PR


def _body(sp_ref, w_ref, x_ref, act_ref, gain_ref, thr_ref, freq_ref,
          ns_ref, act_o_ref, thr_o_ref, gain_o_ref, freq_o_ref, acc_ref):
    i = pl.program_id(0)

    @pl.when(i == 0)
    def _():
        acc_ref[...] = jnp.zeros_like(acc_ref)

    acc_ref[...] += jnp.dot(sp_ref[...], w_ref[...],
                            preferred_element_type=jnp.float32)

    @pl.when(i == _STEPS - 1)
    def _():
        lat = acc_ref[...]
        gain = gain_ref[...]
        gain = gain + (1.0 - gain) * 0.2
        xt = x_ref[...] + lat
        act = _BETA * act_ref[...] + xt * gain + 0.05
        thr = thr_ref[...]
        ns = act > thr
        nsf = ns.astype(jnp.float32)
        freq = _FREQ_BETA * freq_ref[...] + (1.0 - _FREQ_BETA) * nsf
        thr = jnp.where(freq > _TARGET_FREQUENCY, thr + 0.05, thr)
        thr = jnp.where(freq < _TARGET_FREQUENCY, thr / 1.05, thr)
        gain = jnp.where(ns, _REFRACTORY_INPUT_GAIN, gain)
        act = jnp.where(ns, 0.0, act)
        ns_ref[...] = nsf
        act_o_ref[...] = act
        thr_o_ref[...] = thr
        gain_o_ref[...] = gain
        freq_o_ref[...] = freq


@jax.jit
def _step(sp, weights, x, act, gain, thr, freq):
    flat = jax.ShapeDtypeStruct((1, _N), jnp.float32)
    state_spec = pl.BlockSpec((1, _N), lambda i: (0, 0))
    return pl.pallas_call(
        _body,
        grid=(_STEPS,),
        in_specs=[
            pl.BlockSpec((1, _PR), lambda i: (0, i)),
            pl.BlockSpec((_PR, _N), lambda i: (i, 0)),
            state_spec, state_spec, state_spec, state_spec, state_spec,
        ],
        out_specs=(state_spec,) * 5,
        out_shape=(flat,) * 5,
        scratch_shapes=[pltpu.VMEM((1, _N), jnp.float32)],
    )(sp, weights, x, act, gain, thr, freq)


def kernel(x, activation, input_gain, spikes, threshold, freq_act,
           lateral_weights):
    shape = x.shape
    sp = spikes.reshape(1, _N).astype(jnp.float32)
    nsf, act, thr, gain, freq = _step(
        sp,
        lateral_weights,
        x.reshape(1, _N),
        activation.reshape(1, _N),
        input_gain.reshape(1, _N),
        threshold.reshape(1, _N),
        freq_act.reshape(1, _N),
    )
    return (
        nsf.reshape(shape).astype(bool),
        act.reshape(shape),
        thr.reshape(shape),
        gain.reshape(shape),
        freq.reshape(shape),
    )


# speedup vs baseline: 8.5237x; 1.0452x over previous
import jax
import jax.numpy as jnp
from jax.experimental import pallas as pl
from jax.experimental.pallas import tpu as pltpu

_BETA = 0.9
_FREQ_BETA = 0.95
_TARGET_FREQUENCY = 0.1
_REFRACTORY_INPUT_GAIN = -0.3

_N = 4096
_PR = 512
_STEPS = _N // _PR


def _body(sp_ref, w_ref, x_ref, act_ref, gain_ref, thr_ref, freq_ref,
          ns_ref, act_o_ref, thr_o_ref, gain_o_ref, freq_o_ref, acc_ref):
    i = pl.program_id(0)

    @pl.when(i == 0)
    def _():
        acc_ref[...] = jnp.zeros_like(acc_ref)

    acc_ref[...] += jnp.dot(sp_ref[...], w_ref[...],
                            preferred_element_type=jnp.float32)

    @pl.when(i == _STEPS - 1)
    def _():
        lat = acc_ref[...]
        gain = gain_ref[...]
        gain = gain + (1.0 - gain) * 0.2
        xt = x_ref[...] + lat
        act = _BETA * act_ref[...] + xt * gain + 0.05
        thr = thr_ref[...]
        ns = act > thr
        nsf = ns.astype(jnp.float32)
        freq = _FREQ_BETA * freq_ref[...] + (1.0 - _FREQ_BETA) * nsf
        thr = jnp.where(freq > _TARGET_FREQUENCY, thr + 0.05, thr)
        thr = jnp.where(freq < _TARGET_FREQUENCY, thr / 1.05, thr)
        gain = jnp.where(ns, _REFRACTORY_INPUT_GAIN, gain)
        act = jnp.where(ns, 0.0, act)
        ns_ref[...] = nsf
        act_o_ref[...] = act
        thr_o_ref[...] = thr
        gain_o_ref[...] = gain
        freq_o_ref[...] = freq


@jax.jit
def _step(sp, weights, x, act, gain, thr, freq):
    flat = jax.ShapeDtypeStruct((1, _N), jnp.float32)
    state_spec = pl.BlockSpec((1, _N), lambda i: (0, 0))
    return pl.pallas_call(
        _body,
        grid=(_STEPS,),
        in_specs=[
            pl.BlockSpec((1, _PR), lambda i: (0, i)),
            pl.BlockSpec((_PR, _N), lambda i: (i, 0)),
            state_spec, state_spec, state_spec, state_spec, state_spec,
        ],
        out_specs=(state_spec,) * 5,
        out_shape=(flat,) * 5,
        scratch_shapes=[pltpu.VMEM((1, _N), jnp.float32)],
    )(sp, weights, x, act, gain, thr, freq)


def kernel(x, activation, input_gain, spikes, threshold, freq_act,
           lateral_weights):
    shape = x.shape
    sp = spikes.reshape(1, _N).astype(jnp.float32)
    nsf, act, thr, gain, freq = _step(
        sp,
        lateral_weights,
        x.reshape(1, _N),
        activation.reshape(1, _N),
        input_gain.reshape(1, _N),
        threshold.reshape(1, _N),
        freq_act.reshape(1, _N),
    )
    return (
        nsf.reshape(shape).astype(bool),
        act.reshape(shape),
        thr.reshape(shape),
        gain.reshape(shape),
        freq.reshape(shape),
    )
